# baseline (speedup 1.0000x reference)
import functools

import jax
import jax.numpy as jnp
from jax import lax
from jax.experimental import pallas as pl
from jax.experimental.pallas import tpu as pltpu
from jax.experimental.pallas import tpu_sc as plsc

_C = 0.1
_D = 64
_L = 16
_NC = 2
_NS = 16
_NW = _NC * _NS
_CH = 400


def _rsqrt16(x):
    bits = plsc.bitcast(x, jnp.int32)
    seed = jnp.int32(0x5F3759DF) - lax.shift_right_logical(bits, jnp.int32(1))
    y = plsc.bitcast(seed, jnp.float32)
    half_x = x * 0.5
    y = y * (1.5 - half_x * y * y)
    y = y * (1.5 - half_x * y * y)
    return y


def _sc_body(ids_hbm, table_hbm, s16_hbm, c16_hbm, out_hbm,
             idx_a, idx_b, gbuf_a, gbuf_b, obuf_a, obuf_b,
             s_v, c_v, gs_a, gs_b, ws_a, ws_b):
    wid = lax.axis_index("s") * _NC + lax.axis_index("c")
    n_rows = ids_hbm.shape[0] // _NW
    n_chunks = n_rows // _CH
    n_groups = _CH // _L
    base = wid * n_rows

    pltpu.sync_copy(s16_hbm, s_v)
    pltpu.sync_copy(c16_hbm, c_v)
    sv = s_v[...]
    cv = c_v[...]
    sig = 1.0 / (1.0 + jnp.exp(-cv))
    mul = sv * sig
    coef = _C * sv * sv

    idxs = (idx_a, idx_b)
    gbufs = (gbuf_a, gbuf_b)
    obufs = (obuf_a, obuf_b)
    gsems = (gs_a, gs_b)
    wsems = (ws_a, ws_b)

    def fire(g, b):
        off = base + g * _CH
        pltpu.sync_copy(ids_hbm.at[pl.ds(off, _CH)], idxs[b])
        pltpu.async_copy(table_hbm.at[idxs[b]], gbufs[b], gsems[b])

    def wait_gather(b):
        pltpu.make_async_copy(table_hbm.at[idxs[b]], gbufs[b], gsems[b]).wait()

    def put(g, b):
        off = base + g * _CH
        pltpu.async_copy(obufs[b], out_hbm.at[pl.ds(off, _CH)], wsems[b])

    def wait_put(b):
        pltpu.make_async_copy(
            obufs[b], out_hbm.at[pl.ds(base, _CH)], wsems[b]).wait()

    def compute(src, dst):
        @plsc.parallel_loop(0, n_groups, unroll=1)
        def group(j):
            ridx = j * _L + lax.iota(jnp.int32, 16)
            accs = [jnp.zeros((_L,), jnp.float32) for _ in range(4)]
            for c in range(_D):
                cvec = jnp.full((_L,), c, jnp.int32)
                v = plsc.load_gather(src, [ridx, cvec])
                accs[c % 4] = accs[c % 4] + v * v
            ns = (accs[0] + accs[1]) + (accs[2] + accs[3])
            t = 1.0 + coef * ns
            r = _rsqrt16(t)
            fac = mul / (1.0 + t * r)
            for c in range(_D):
                cvec = jnp.full((_L,), c, jnp.int32)
                v = plsc.load_gather(src, [ridx, cvec])
                plsc.store_scatter(dst, [ridx, cvec], v * fac)

    fire(0, 0)

    def pair_body(i, carry):
        g0 = 2 * i
        g1 = g0 + 1
        fire(g1, 1)

        @pl.when(i >= 1)
        def _():
            wait_put(0)
        wait_gather(0)
        compute(gbufs[0], obufs[0])
        put(g0, 0)

        @pl.when(g1 + 1 < n_chunks)
        def _():
            fire(g1 + 1, 0)

        @pl.when(i >= 1)
        def _():
            wait_put(1)
        wait_gather(1)
        compute(gbufs[1], obufs[1])
        put(g1, 1)
        return carry

    lax.fori_loop(0, n_chunks // 2, pair_body, 0)
    wait_put(0)
    wait_put(1)


def kernel(input_ids, embed_table, scale, curvature_param):
    n_tok = input_ids.shape[0] * input_ids.shape[1]
    ids = input_ids.reshape(n_tok).astype(jnp.int32)
    s16 = jnp.broadcast_to(scale.astype(jnp.float32), (_L,))
    c16 = jnp.broadcast_to(curvature_param.astype(jnp.float32), (_L,))

    sc_call = functools.partial(
        pl.kernel,
        out_type=jax.ShapeDtypeStruct((n_tok, _D), jnp.float32),
        mesh=plsc.VectorSubcoreMesh(core_axis_name="c", subcore_axis_name="s"),
        compiler_params=pltpu.CompilerParams(
            needs_layout_passes=False, use_tc_tiling_on_sc=False),
        scratch_types=[
            pltpu.VMEM((_CH,), jnp.int32),
            pltpu.VMEM((_CH,), jnp.int32),
            pltpu.VMEM((_CH, _D), jnp.float32),
            pltpu.VMEM((_CH, _D), jnp.float32),
            pltpu.VMEM((_CH, _D), jnp.float32),
            pltpu.VMEM((_CH, _D), jnp.float32),
            pltpu.VMEM((_L,), jnp.float32),
            pltpu.VMEM((_L,), jnp.float32),
            pltpu.SemaphoreType.DMA,
            pltpu.SemaphoreType.DMA,
            pltpu.SemaphoreType.DMA,
            pltpu.SemaphoreType.DMA,
        ],
    )(_sc_body)
    out = sc_call(ids, embed_table, s16, c16)
    return out.reshape(input_ids.shape[0], input_ids.shape[1], _D)


# speedup vs baseline: 2.4292x; 2.4292x over previous
import functools

import jax
import jax.numpy as jnp
from jax import lax
from jax.experimental import pallas as pl
from jax.experimental.pallas import tpu as pltpu
from jax.experimental.pallas import tpu_sc as plsc

_C = 0.1
_D = 64
_L = 16
_NC = 2
_NS = 16
_NW = _NC * _NS
_CH = 400


def _rsqrt16(x):
    bits = plsc.bitcast(x, jnp.int32)
    seed = jnp.int32(0x5F3759DF) - lax.shift_right_logical(bits, jnp.int32(1))
    y = plsc.bitcast(seed, jnp.float32)
    half_x = x * 0.5
    y = y * (1.5 - half_x * y * y)
    y = y * (1.5 - half_x * y * y)
    return y


def _sc_body(ids_hbm, table_hbm, s16_hbm, c16_hbm, out_hbm,
             idx_a, idx_b, gbuf_a, gbuf_b, obuf_a, obuf_b,
             s_v, c_v, pbuf, gs_a, gs_b, ws_a, ws_b):
    wid = lax.axis_index("s") * _NC + lax.axis_index("c")
    n_rows = ids_hbm.shape[0] // _NW
    n_chunks = n_rows // _CH
    n_groups = _CH // _L
    base = wid * n_rows

    pltpu.sync_copy(s16_hbm, s_v)
    pltpu.sync_copy(c16_hbm, c_v)
    sv = s_v[...]
    cv = c_v[...]
    sig = 1.0 / (1.0 + jnp.exp(-cv))
    mul = sv * sig
    coef = _C * sv * sv

    idxs = (idx_a, idx_b)
    gbufs = (gbuf_a, gbuf_b)
    obufs = (obuf_a, obuf_b)
    gsems = (gs_a, gs_b)
    wsems = (ws_a, ws_b)

    def fire(g, b):
        off = base + g * _CH
        pltpu.sync_copy(ids_hbm.at[pl.ds(off, _CH)], idxs[b])
        pltpu.async_copy(table_hbm.at[idxs[b]], gbufs[b], gsems[b])

    def wait_gather(b):
        pltpu.make_async_copy(table_hbm.at[idxs[b]], gbufs[b], gsems[b]).wait()

    def put(g, b):
        off = base + g * _CH
        pltpu.async_copy(obufs[b], out_hbm.at[pl.ds(off, _CH)], wsems[b])

    def wait_put(b):
        pltpu.make_async_copy(
            obufs[b], out_hbm.at[pl.ds(base, _CH)], wsems[b]).wait()

    def compute(src, dst, pbuf):
        @plsc.parallel_loop(0, n_groups, unroll=1)
        def group(j):
            for k in range(_L):
                i = j * _L + k
                p = jnp.zeros((_L,), jnp.float32)
                for q in range(4):
                    v = src[i, pl.ds(q * _L, _L)]
                    p = p + v * v
                pbuf[j, k, pl.ds(0, _L)] = p
            lane = lax.iota(jnp.int32, 16)
            jv = jnp.full((_L,), 0, jnp.int32) + j
            ns = jnp.zeros((_L,), jnp.float32)
            for c in range(_L):
                cvec = jnp.full((_L,), c, jnp.int32)
                ns = ns + plsc.load_gather(pbuf, [jv, lane, cvec])
            t = 1.0 + coef * ns
            r = _rsqrt16(t)
            fac = mul / (1.0 + t * r)
            for k in range(_L):
                i = j * _L + k
                f = jnp.broadcast_to(fac[k], (_L,))
                for q in range(4):
                    dst[i, pl.ds(q * _L, _L)] = src[i, pl.ds(q * _L, _L)] * f

    fire(0, 0)

    def pair_body(i, carry):
        g0 = 2 * i
        g1 = g0 + 1
        fire(g1, 1)

        @pl.when(i >= 1)
        def _():
            wait_put(0)
        wait_gather(0)
        compute(gbufs[0], obufs[0], pbuf)
        put(g0, 0)

        @pl.when(g1 + 1 < n_chunks)
        def _():
            fire(g1 + 1, 0)

        @pl.when(i >= 1)
        def _():
            wait_put(1)
        wait_gather(1)
        compute(gbufs[1], obufs[1], pbuf)
        put(g1, 1)
        return carry

    lax.fori_loop(0, n_chunks // 2, pair_body, 0)
    wait_put(0)
    wait_put(1)


def kernel(input_ids, embed_table, scale, curvature_param):
    n_tok = input_ids.shape[0] * input_ids.shape[1]
    ids = input_ids.reshape(n_tok).astype(jnp.int32)
    s16 = jnp.broadcast_to(scale.astype(jnp.float32), (_L,))
    c16 = jnp.broadcast_to(curvature_param.astype(jnp.float32), (_L,))

    sc_call = functools.partial(
        pl.kernel,
        out_type=jax.ShapeDtypeStruct((n_tok, _D), jnp.float32),
        mesh=plsc.VectorSubcoreMesh(core_axis_name="c", subcore_axis_name="s"),
        compiler_params=pltpu.CompilerParams(
            needs_layout_passes=False, use_tc_tiling_on_sc=False),
        scratch_types=[
            pltpu.VMEM((_CH,), jnp.int32),
            pltpu.VMEM((_CH,), jnp.int32),
            pltpu.VMEM((_CH, _D), jnp.float32),
            pltpu.VMEM((_CH, _D), jnp.float32),
            pltpu.VMEM((_CH, _D), jnp.float32),
            pltpu.VMEM((_CH, _D), jnp.float32),
            pltpu.VMEM((_L,), jnp.float32),
            pltpu.VMEM((_L,), jnp.float32),
            pltpu.VMEM((_CH // _L, _L, 17), jnp.float32),
            pltpu.SemaphoreType.DMA,
            pltpu.SemaphoreType.DMA,
            pltpu.SemaphoreType.DMA,
            pltpu.SemaphoreType.DMA,
        ],
    )(_sc_body)
    out = sc_call(ids, embed_table, s16, c16)
    return out.reshape(input_ids.shape[0], input_ids.shape[1], _D)
